# Initial kernel scaffold; baseline (speedup 1.0000x reference)
import jax
import jax.numpy as jnp
from jax.experimental import pallas as pl


def kernel(x, edge_index, batch, W1, b1, W2, b2, Wfc, bfc):
    raise NotImplementedError("write your pallas kernel here")


# speedup vs baseline: 11.6943x; 11.6943x over previous
import functools

import jax
import jax.numpy as jnp
from jax import lax
from jax.experimental import pallas as pl
from jax.experimental.pallas import tpu as pltpu
from jax.experimental.pallas import tpu_sc as plsc

_N = 10000
_E = 320000
_B = 64
_D = 128

_NC = 2
_NS = 16
_L = 16

_K = 80
_EPT = _E // (_NC * _NS)
_NCHUNK = _EPT // _K
_ZCH = _N // _K
_ZITER = (_ZCH + _NS - 1) // _NS

_R = 400
_G = _N // _R


def _sc_mesh():
    return plsc.VectorSubcoreMesh(
        core_axis_name="c", subcore_axis_name="s",
        num_cores=_NC, num_subcores=_NS)


@functools.cache
def _sc_degree_fn():
  return functools.partial(
      pl.kernel,
      out_type=jax.ShapeDtypeStruct((_NC * _N, _D), jnp.float32),
      mesh=_sc_mesh(),
      scratch_types=[
          pltpu.VMEM((_K,), jnp.int32),
          pltpu.VMEM((_K, _D), jnp.float32),
          pltpu.VMEM_SHARED((_N, _D), jnp.float32),
      ],
  )(_sc_degree_body)


def _sc_degree(dst, zeros_l, ones_l):
    return _sc_degree_fn()(dst, zeros_l, ones_l)


def _sc_degree_body(dst_hbm, zeros_hbm, ones_hbm, out_hbm, dstv, rows, acc):
    c = lax.axis_index("c")
    s = lax.axis_index("s")
    tile = c * _NS + s

    pltpu.sync_copy(zeros_hbm, rows)

    def zacc(g, carry):
        cidx = s + g * _NS

        @pl.when(cidx < _ZCH)
        def _():
            pltpu.sync_copy(rows, acc.at[pl.ds(cidx * _K, _K)])

        return carry

    lax.fori_loop(0, _ZITER, zacc, 0)

    pltpu.sync_copy(ones_hbm, rows)
    plsc.subcore_barrier()

    ebase = tile * _EPT

    def step(i, carry):
        pltpu.sync_copy(dst_hbm.at[pl.ds(ebase + i * _K, _K)], dstv)
        pltpu.sync_copy(rows, acc.at[dstv], add=True)
        return carry

    lax.fori_loop(0, _NCHUNK, step, 0)
    plsc.subcore_barrier()

    def flush(g, carry):
        cidx = s + g * _NS

        @pl.when(cidx < _ZCH)
        def _():
            pltpu.sync_copy(acc.at[pl.ds(cidx * _K, _K)], rows)
            pltpu.sync_copy(rows, out_hbm.at[pl.ds(c * _N + cidx * _K, _K)])

        return carry

    lax.fori_loop(0, _ZITER, flush, 0)


@functools.cache
def _sc_scatter_fn():
  return functools.partial(
      pl.kernel,
      out_type=jax.ShapeDtypeStruct((_NC * _N, _D), jnp.float32),
      mesh=_sc_mesh(),
      scratch_types=[
          pltpu.VMEM((_K,), jnp.int32),
          pltpu.VMEM((_K,), jnp.int32),
          pltpu.VMEM((_K, _D), jnp.float32),
          pltpu.VMEM_SHARED((_N, _D), jnp.float32),
          pltpu.SemaphoreType.DMA,
      ],
  )(_sc_scatter_body)


def _sc_scatter(h, src, dst, zeros_d):
    return _sc_scatter_fn()(h, src, dst, zeros_d)


def _sc_scatter_body(h_hbm, src_hbm, dst_hbm, zeros_hbm, out_hbm,
                     srcv, dstv, rows, acc, sem):
    c = lax.axis_index("c")
    s = lax.axis_index("s")
    tile = c * _NS + s

    pltpu.sync_copy(zeros_hbm, rows)

    def zacc(g, carry):
        cidx = s + g * _NS

        @pl.when(cidx < _ZCH)
        def _():
            pltpu.sync_copy(rows, acc.at[pl.ds(cidx * _K, _K)])

        return carry

    lax.fori_loop(0, _ZITER, zacc, 0)
    plsc.subcore_barrier()

    ebase = tile * _EPT

    def step(i, carry):
        pltpu.sync_copy(src_hbm.at[pl.ds(ebase + i * _K, _K)], srcv)
        pltpu.sync_copy(dst_hbm.at[pl.ds(ebase + i * _K, _K)], dstv)
        pltpu.async_copy(h_hbm.at[srcv], rows, sem).wait()
        pltpu.sync_copy(rows, acc.at[dstv], add=True)
        return carry

    lax.fori_loop(0, _NCHUNK, step, 0)
    plsc.subcore_barrier()

    def flush(g, carry):
        cidx = s + g * _NS

        @pl.when(cidx < _ZCH)
        def _():
            pltpu.sync_copy(acc.at[pl.ds(cidx * _K, _K)], rows)
            pltpu.sync_copy(rows, out_hbm.at[pl.ds(c * _N + cidx * _K, _K)])

        return carry

    lax.fori_loop(0, _ZITER, flush, 0)


def _tc_layer1_body(x_ref, d0_ref, d1_ref, w_ref, h1p_ref, dinv_ref):
    deg = d0_ref[...] + d1_ref[...] + 1.0
    dinv = lax.rsqrt(deg)
    h = jnp.dot(x_ref[...], w_ref[...], preferred_element_type=jnp.float32)
    dinv_ref[...] = dinv
    h1p_ref[...] = h * dinv


def _tc_layer1(x, deg0, deg1, W1):
    return pl.pallas_call(
        _tc_layer1_body,
        grid=(_G,),
        in_specs=[
            pl.BlockSpec((_R, _D), lambda i: (i, 0)),
            pl.BlockSpec((_R, _D), lambda i: (i, 0)),
            pl.BlockSpec((_R, _D), lambda i: (i, 0)),
            pl.BlockSpec((_D, _D), lambda i: (0, 0)),
        ],
        out_specs=[
            pl.BlockSpec((_R, _D), lambda i: (i, 0)),
            pl.BlockSpec((_R, _D), lambda i: (i, 0)),
        ],
        out_shape=[
            jax.ShapeDtypeStruct((_N, _D), jnp.float32),
            jax.ShapeDtypeStruct((_N, _D), jnp.float32),
        ],
    )(x, deg0, deg1, W1)


def _tc_layer2_body(s0_ref, s1_ref, h1p_ref, dinv_ref, w_ref, b_ref, h2p_ref):
    dinv = dinv_ref[...]
    z1 = jnp.maximum(
        (s0_ref[...] + s1_ref[...] + h1p_ref[...]) * dinv + b_ref[...], 0.0)
    h2p_ref[...] = jnp.dot(
        z1, w_ref[...], preferred_element_type=jnp.float32) * dinv


def _tc_layer2(s0, s1, h1p, dinv, W2, b1):
    return pl.pallas_call(
        _tc_layer2_body,
        grid=(_G,),
        in_specs=[
            pl.BlockSpec((_R, _D), lambda i: (i, 0)),
            pl.BlockSpec((_R, _D), lambda i: (i, 0)),
            pl.BlockSpec((_R, _D), lambda i: (i, 0)),
            pl.BlockSpec((_R, _D), lambda i: (i, 0)),
            pl.BlockSpec((_D, _D), lambda i: (0, 0)),
            pl.BlockSpec((1, _D), lambda i: (0, 0)),
        ],
        out_specs=pl.BlockSpec((_R, _D), lambda i: (i, 0)),
        out_shape=jax.ShapeDtypeStruct((_N, _D), jnp.float32),
    )(s0, s1, h1p, dinv, W2, b1.reshape(1, _D))


def _tc_pool_body(s0_ref, s1_ref, h2p_ref, dinv_ref, b_ref, batch_ref,
                  wfct_ref, bfc_ref, out_ref, sum_sc, cnt_sc):
    i = pl.program_id(0)
    z2 = jnp.maximum(
        (s0_ref[...] + s1_ref[...] + h2p_ref[...]) * dinv_ref[...]
        + b_ref[...], 0.0)
    gid = lax.broadcasted_iota(jnp.int32, (_R, _B), 1)
    onehot = (batch_ref[...] == gid).astype(jnp.float32)
    sums = lax.dot_general(onehot, z2, (((0,), (0,)), ((), ())),
                           preferred_element_type=jnp.float32)
    cnts = lax.dot_general(onehot, jnp.ones((_R, _D), jnp.float32),
                           (((0,), (0,)), ((), ())),
                           preferred_element_type=jnp.float32)

    @pl.when(i == 0)
    def _():
        sum_sc[...] = jnp.zeros((_B, _D), jnp.float32)
        cnt_sc[...] = jnp.zeros((_B, _D), jnp.float32)

    sum_sc[...] += sums
    cnt_sc[...] += cnts

    @pl.when(i == _G - 1)
    def _():
        pooled = sum_sc[...] / jnp.maximum(cnt_sc[...], 1.0)
        out_ref[...] = jnp.dot(
            pooled, wfct_ref[...], preferred_element_type=jnp.float32
        ) + bfc_ref[...]


def _tc_pool_fc(s0, s1, h2p, dinv, b2, batch, WfcT, bfc):
    return pl.pallas_call(
        _tc_pool_body,
        grid=(_G,),
        in_specs=[
            pl.BlockSpec((_R, _D), lambda i: (i, 0)),
            pl.BlockSpec((_R, _D), lambda i: (i, 0)),
            pl.BlockSpec((_R, _D), lambda i: (i, 0)),
            pl.BlockSpec((_R, _D), lambda i: (i, 0)),
            pl.BlockSpec((1, _D), lambda i: (0, 0)),
            pl.BlockSpec((_R, 1), lambda i: (i, 0)),
            pl.BlockSpec((_D, _D), lambda i: (0, 0)),
            pl.BlockSpec((1, _D), lambda i: (0, 0)),
        ],
        out_specs=pl.BlockSpec((_B, _D), lambda i: (0, 0)),
        out_shape=jax.ShapeDtypeStruct((_B, _D), jnp.float32),
        scratch_shapes=[
            pltpu.VMEM((_B, _D), jnp.float32),
            pltpu.VMEM((_B, _D), jnp.float32),
        ],
    )(s0, s1, h2p, dinv, b2.reshape(1, _D), batch.reshape(_N, 1),
      WfcT, bfc.reshape(1, _D))


def kernel(x, edge_index, batch, W1, b1, W2, b2, Wfc, bfc):
    src = edge_index[0]
    dst = edge_index[1]

    zeros_d = jnp.zeros((_K, _D), jnp.float32)
    ones_d = jnp.ones((_K, _D), jnp.float32)

    degp = _sc_degree(dst, zeros_d, ones_d)
    deg0, deg1 = degp[:_N], degp[_N:]

    h1p, dinv = _tc_layer1(x, deg0, deg1, W1)

    s1 = _sc_scatter(h1p, src, dst, zeros_d)
    h2p = _tc_layer2(s1[:_N], s1[_N:], h1p, dinv, W2, b1)

    s2 = _sc_scatter(h2p, src, dst, zeros_d)
    return _tc_pool_fc(s2[:_N], s2[_N:], h2p, dinv, b2, batch, Wfc.T, bfc)
